# Initial kernel scaffold; baseline (speedup 1.0000x reference)
import functools

import jax
import jax.numpy as jnp
from jax import lax
from jax.experimental import pallas as pl
from jax.experimental.pallas import tpu as pltpu, tpu_sc as plsc

_NC = 2
_NS = 16
_NW = _NC * _NS
_LANES = 16
_CHUNK = 64


def _trig_body(theta_ref, cs_ref):
    th = theta_ref[...]
    h = th.shape[1]
    cs_ref[:, :h] = jnp.cos(th)
    cs_ref[:, h:] = jnp.sin(th)


def _make_cs_table(rel_emb):
    r, h = rel_emb.shape
    return pl.pallas_call(
        _trig_body,
        out_shape=jax.ShapeDtypeStruct((r, 2 * h), jnp.float32),
    )(rel_emb)


def _sc_score(head_index, rel_type, tail_index, node_emb, node_emb_im, cs_tab):
    batch = head_index.shape[0]
    hidden = node_emb.shape[1]
    nslice = hidden // _LANES
    per_w = batch // _NW
    n_chunks = per_w // _CHUNK
    mesh = plsc.VectorSubcoreMesh(
        core_axis_name="c", subcore_axis_name="s",
        num_cores=_NC, num_subcores=_NS,
    )

    @functools.partial(
        pl.kernel,
        out_type=jax.ShapeDtypeStruct((batch,), jnp.float32),
        mesh=mesh,
        scratch_types=[
            pltpu.VMEM((_CHUNK,), jnp.int32),
            pltpu.VMEM((_CHUNK,), jnp.int32),
            pltpu.VMEM((_CHUNK,), jnp.int32),
            pltpu.VMEM((_CHUNK, hidden), jnp.float32),
            pltpu.VMEM((_CHUNK, hidden), jnp.float32),
            pltpu.VMEM((_CHUNK, hidden), jnp.float32),
            pltpu.VMEM((_CHUNK, hidden), jnp.float32),
            pltpu.VMEM((_CHUNK, 2 * hidden), jnp.float32),
            pltpu.VMEM((_CHUNK * _LANES,), jnp.float32),
            pltpu.VMEM((per_w,), jnp.float32),
            pltpu.SemaphoreType.DMA,
        ],
    )
    def score_kernel(head_hbm, rel_hbm, tail_hbm, emb_hbm, embim_hbm, cs_hbm,
                     out_hbm, idx_h, idx_r, idx_t, hre, him, tre, tim, cs,
                     accs, score, sem):
        wid = lax.axis_index("s") * _NC + lax.axis_index("c")
        base = wid * per_w
        lane_ids = lax.iota(jnp.int32, _LANES) * _LANES

        for c in range(n_chunks):
            cbase = base + c * _CHUNK
            pltpu.sync_copy(head_hbm.at[pl.ds(cbase, _CHUNK)], idx_h)
            pltpu.sync_copy(rel_hbm.at[pl.ds(cbase, _CHUNK)], idx_r)
            pltpu.sync_copy(tail_hbm.at[pl.ds(cbase, _CHUNK)], idx_t)
            cps = (
                pltpu.async_copy(emb_hbm.at[idx_h], hre, sem),
                pltpu.async_copy(embim_hbm.at[idx_h], him, sem),
                pltpu.async_copy(emb_hbm.at[idx_t], tre, sem),
                pltpu.async_copy(embim_hbm.at[idx_t], tim, sem),
                pltpu.async_copy(cs_hbm.at[idx_r], cs, sem),
            )
            for cp in cps:
                cp.wait()

            def tbody(t, carry):
                acc = jnp.zeros((_LANES,), jnp.float32)
                for j in range(nslice):
                    sl = pl.ds(j * _LANES, _LANES)
                    cv = cs[t, sl]
                    sv = cs[t, pl.ds(hidden + j * _LANES, _LANES)]
                    a = hre[t, sl]
                    b = him[t, sl]
                    u = tre[t, sl]
                    v = tim[t, sl]
                    re = cv * a - sv * b - u
                    im = cv * b + sv * a - v
                    acc = acc + (re * re + im * im)
                accs[pl.ds(t * _LANES, _LANES)] = acc
                return carry

            lax.fori_loop(0, _CHUNK, tbody, 0)

            for g in range(_CHUNK // _LANES):
                s2 = plsc.load_gather(accs, [lane_ids + g * _LANES * _LANES])
                for k in range(1, _LANES):
                    s2 = s2 + plsc.load_gather(
                        accs, [lane_ids + (g * _LANES * _LANES + k)])
                x = jnp.maximum(s2, jnp.float32(1e-12))
                bits = plsc.bitcast(x, jnp.int32)
                bits = jnp.int32(0x5F3759DF) - lax.shift_right_logical(bits, 1)
                y = plsc.bitcast(bits, jnp.float32)
                for _ in range(3):
                    y = y * (jnp.float32(1.5) - jnp.float32(0.5) * x * y * y)
                score[pl.ds(c * _CHUNK + g * _LANES, _LANES)] = -(x * y)

        pltpu.sync_copy(score, out_hbm.at[pl.ds(base, per_w)])

    return score_kernel(head_index, rel_type, tail_index,
                        node_emb, node_emb_im, cs_tab)


def kernel(head_index, rel_type, tail_index, node_emb, node_emb_im, rel_emb):
    cs_tab = _make_cs_table(rel_emb)
    return _sc_score(head_index, rel_type, tail_index,
                     node_emb, node_emb_im, cs_tab)


# speedup vs baseline: 1.7068x; 1.7068x over previous
import functools

import jax
import jax.numpy as jnp
from jax import lax
from jax.experimental import pallas as pl
from jax.experimental.pallas import tpu as pltpu, tpu_sc as plsc

_NC = 2
_NS = 16
_NW = _NC * _NS
_LANES = 16
_CHUNK = 64


def _trig_body(theta_ref, cs_ref):
    th = theta_ref[...]
    h = th.shape[1]
    cs_ref[:, :h] = jnp.cos(th)
    cs_ref[:, h:] = jnp.sin(th)


def _make_cs_table(rel_emb):
    r, h = rel_emb.shape
    return pl.pallas_call(
        _trig_body,
        out_shape=jax.ShapeDtypeStruct((r, 2 * h), jnp.float32),
    )(rel_emb)


def _sc_score(head_index, rel_type, tail_index, node_emb, node_emb_im, cs_tab):
    batch = head_index.shape[0]
    hidden = node_emb.shape[1]
    nslice = hidden // _LANES
    per_w = batch // _NW
    n_chunks = per_w // _CHUNK
    mesh = plsc.VectorSubcoreMesh(
        core_axis_name="c", subcore_axis_name="s",
        num_cores=_NC, num_subcores=_NS,
    )

    @functools.partial(
        pl.kernel,
        out_type=jax.ShapeDtypeStruct((batch,), jnp.float32),
        mesh=mesh,
        scratch_types=[
            pltpu.VMEM((_CHUNK,), jnp.int32),
            pltpu.VMEM((_CHUNK,), jnp.int32),
            pltpu.VMEM((_CHUNK,), jnp.int32),
            pltpu.VMEM((_CHUNK, hidden), jnp.float32),
            pltpu.VMEM((_CHUNK, hidden), jnp.float32),
            pltpu.VMEM((_CHUNK, hidden), jnp.float32),
            pltpu.VMEM((_CHUNK, hidden), jnp.float32),
            pltpu.VMEM((_CHUNK, 2 * hidden), jnp.float32),
            pltpu.VMEM((per_w,), jnp.float32),
            pltpu.SemaphoreType.DMA,
        ],
    )
    def score_kernel(head_hbm, rel_hbm, tail_hbm, emb_hbm, embim_hbm, cs_hbm,
                     out_hbm, idx_h, idx_r, idx_t, hre, him, tre, tim, cs,
                     score, sem):
        wid = lax.axis_index("s") * _NC + lax.axis_index("c")
        base = wid * per_w
        lane_iota = lax.iota(jnp.int32, _LANES)
        perm_idx = [lax.iota(jnp.int32, _LANES) ^ jnp.int32(d)
                    for d in (1, 2, 4, 8)]
        gdims = lax.GatherDimensionNumbers(
            offset_dims=(), collapsed_slice_dims=(0,), start_index_map=(0,))

        def _lperm(x, pidx):
            return lax.gather(x, pidx[:, None], gdims, (1,),
                              mode=lax.GatherScatterMode.PROMISE_IN_BOUNDS)

        for c in range(n_chunks):
            cbase = base + c * _CHUNK
            pltpu.sync_copy(head_hbm.at[pl.ds(cbase, _CHUNK)], idx_h)
            pltpu.sync_copy(rel_hbm.at[pl.ds(cbase, _CHUNK)], idx_r)
            pltpu.sync_copy(tail_hbm.at[pl.ds(cbase, _CHUNK)], idx_t)
            cps = (
                pltpu.async_copy(emb_hbm.at[idx_h], hre, sem),
                pltpu.async_copy(embim_hbm.at[idx_h], him, sem),
                pltpu.async_copy(emb_hbm.at[idx_t], tre, sem),
                pltpu.async_copy(embim_hbm.at[idx_t], tim, sem),
                pltpu.async_copy(cs_hbm.at[idx_r], cs, sem),
            )
            for cp in cps:
                cp.wait()

            def gbody(g, carry):
                def tbody(t, res):
                    row = g * _LANES + t
                    acc = jnp.zeros((_LANES,), jnp.float32)
                    for j in range(nslice):
                        sl = pl.ds(j * _LANES, _LANES)
                        cv = cs[row, sl]
                        sv = cs[row, pl.ds(hidden + j * _LANES, _LANES)]
                        a = hre[row, sl]
                        b = him[row, sl]
                        u = tre[row, sl]
                        v = tim[row, sl]
                        re = cv * a - sv * b - u
                        im = cv * b + sv * a - v
                        acc = acc + (re * re + im * im)
                    for pidx in perm_idx:
                        acc = acc + _lperm(acc, pidx)
                    return jnp.where(lane_iota == t, acc, res)

                s2 = lax.fori_loop(0, _LANES, tbody,
                                   jnp.zeros((_LANES,), jnp.float32))
                x = jnp.maximum(s2, jnp.float32(1e-12))
                bits = lax.bitcast_convert_type(x, jnp.int32)
                bits = jnp.int32(0x5F3759DF) - lax.shift_right_logical(bits, 1)
                y = lax.bitcast_convert_type(bits, jnp.float32)
                for _ in range(3):
                    y = y * (jnp.float32(1.5) - jnp.float32(0.5) * x * y * y)
                score[pl.ds(c * _CHUNK + g * _LANES, _LANES)] = -(x * y)
                return carry

            lax.fori_loop(0, _CHUNK // _LANES, gbody, 0)

        pltpu.sync_copy(score, out_hbm.at[pl.ds(base, per_w)])

    return score_kernel(head_index, rel_type, tail_index,
                        node_emb, node_emb_im, cs_tab)


def kernel(head_index, rel_type, tail_index, node_emb, node_emb_im, rel_emb):
    cs_tab = _make_cs_table(rel_emb)
    return _sc_score(head_index, rel_type, tail_index,
                     node_emb, node_emb_im, cs_tab)


# speedup vs baseline: 2.5723x; 1.5071x over previous
import functools

import jax
import jax.numpy as jnp
from jax import lax
from jax.experimental import pallas as pl
from jax.experimental.pallas import tpu as pltpu, tpu_sc as plsc

_NC = 2
_NS = 16
_NW = _NC * _NS
_LANES = 16
_CHUNK = 64


def _trig_body(theta_ref, cs_ref):
    th = theta_ref[...]
    h = th.shape[1]
    cs_ref[:, :h] = jnp.cos(th)
    cs_ref[:, h:] = jnp.sin(th)


def _make_cs_table(rel_emb):
    r, h = rel_emb.shape
    return pl.pallas_call(
        _trig_body,
        out_shape=jax.ShapeDtypeStruct((r, 2 * h), jnp.float32),
    )(rel_emb)


def _sc_score(head_index, rel_type, tail_index, node_emb, node_emb_im, cs_tab):
    batch = head_index.shape[0]
    hidden = node_emb.shape[1]
    nslice = hidden // _LANES
    per_w = batch // _NW
    n_chunks = per_w // _CHUNK
    mesh = plsc.VectorSubcoreMesh(
        core_axis_name="c", subcore_axis_name="s",
        num_cores=_NC, num_subcores=_NS,
    )

    @functools.partial(
        pl.kernel,
        out_type=jax.ShapeDtypeStruct((batch,), jnp.float32),
        mesh=mesh,
        scratch_types=[
            pltpu.VMEM((2, _CHUNK), jnp.int32),
            pltpu.VMEM((2, _CHUNK), jnp.int32),
            pltpu.VMEM((2, _CHUNK), jnp.int32),
            pltpu.VMEM((2, _CHUNK, hidden), jnp.float32),
            pltpu.VMEM((2, _CHUNK, hidden), jnp.float32),
            pltpu.VMEM((2, _CHUNK, hidden), jnp.float32),
            pltpu.VMEM((2, _CHUNK, hidden), jnp.float32),
            pltpu.VMEM((2, _CHUNK, 2 * hidden), jnp.float32),
            pltpu.VMEM((per_w,), jnp.float32),
            pltpu.SemaphoreType.DMA,
            pltpu.SemaphoreType.DMA,
            pltpu.SemaphoreType.DMA,
            pltpu.SemaphoreType.DMA,
        ],
    )
    def score_kernel(head_hbm, rel_hbm, tail_hbm, emb_hbm, embim_hbm, cs_hbm,
                     out_hbm, idx_h, idx_r, idx_t, hre, him, tre, tim, cs,
                     score, sem_i0, sem_i1, sem_g0, sem_g1):
        sem_i = (sem_i0, sem_i1)
        sem_g = (sem_g0, sem_g1)
        wid = lax.axis_index("s") * _NC + lax.axis_index("c")
        base = wid * per_w
        lane_iota = lax.iota(jnp.int32, _LANES)
        perm_idx = [lax.iota(jnp.int32, _LANES) ^ jnp.int32(d)
                    for d in (1, 2, 4, 8)]
        gdims = lax.GatherDimensionNumbers(
            offset_dims=(), collapsed_slice_dims=(0,), start_index_map=(0,))

        def _lperm(x, pidx):
            return lax.gather(x, pidx[:, None], gdims, (1,),
                              mode=lax.GatherScatterMode.PROMISE_IN_BOUNDS)

        def fire_idx(c):
            p = c & 1
            cbase = base + c * _CHUNK
            return (
                pltpu.async_copy(head_hbm.at[pl.ds(cbase, _CHUNK)],
                                 idx_h.at[p], sem_i[p]),
                pltpu.async_copy(rel_hbm.at[pl.ds(cbase, _CHUNK)],
                                 idx_r.at[p], sem_i[p]),
                pltpu.async_copy(tail_hbm.at[pl.ds(cbase, _CHUNK)],
                                 idx_t.at[p], sem_i[p]),
            )

        def fire_gather(c):
            p = c & 1
            return (
                pltpu.async_copy(emb_hbm.at[idx_h.at[p]], hre.at[p], sem_g[p]),
                pltpu.async_copy(embim_hbm.at[idx_h.at[p]], him.at[p], sem_g[p]),
                pltpu.async_copy(emb_hbm.at[idx_t.at[p]], tre.at[p], sem_g[p]),
                pltpu.async_copy(embim_hbm.at[idx_t.at[p]], tim.at[p], sem_g[p]),
                pltpu.async_copy(cs_hbm.at[idx_r.at[p]], cs.at[p], sem_g[p]),
            )

        def drain(cps):
            for cp in cps:
                cp.wait()

        idx_cps = {0: fire_idx(0)}
        drain(idx_cps[0])
        gat_cps = {0: fire_gather(0)}
        if n_chunks > 1:
            idx_cps[1] = fire_idx(1)

        for c in range(n_chunks):
            p = c & 1
            if c + 1 < n_chunks:
                drain(idx_cps[c + 1])
                gat_cps[c + 1] = fire_gather(c + 1)
            drain(gat_cps[c])
            if c + 2 < n_chunks:
                idx_cps[c + 2] = fire_idx(c + 2)

            def gbody(g, carry):
                def tbody(t, res):
                    row = g * _LANES + t
                    acc = jnp.zeros((_LANES,), jnp.float32)
                    for j in range(nslice):
                        sl = pl.ds(j * _LANES, _LANES)
                        cv = cs[p, row, sl]
                        sv = cs[p, row, pl.ds(hidden + j * _LANES, _LANES)]
                        a = hre[p, row, sl]
                        b = him[p, row, sl]
                        u = tre[p, row, sl]
                        v = tim[p, row, sl]
                        re = cv * a - sv * b - u
                        im = cv * b + sv * a - v
                        acc = acc + (re * re + im * im)
                    for pidx in perm_idx:
                        acc = acc + _lperm(acc, pidx)
                    return jnp.where(lane_iota == t, acc, res)

                s2 = lax.fori_loop(0, _LANES, tbody,
                                   jnp.zeros((_LANES,), jnp.float32))
                x = jnp.maximum(s2, jnp.float32(1e-12))
                bits = lax.bitcast_convert_type(x, jnp.int32)
                bits = jnp.int32(0x5F3759DF) - lax.shift_right_logical(bits, 1)
                y = lax.bitcast_convert_type(bits, jnp.float32)
                for _ in range(3):
                    y = y * (jnp.float32(1.5) - jnp.float32(0.5) * x * y * y)
                score[pl.ds(c * _CHUNK + g * _LANES, _LANES)] = -(x * y)
                return carry

            lax.fori_loop(0, _CHUNK // _LANES, gbody, 0)

        pltpu.sync_copy(score, out_hbm.at[pl.ds(base, per_w)])

    return score_kernel(head_index, rel_type, tail_index,
                        node_emb, node_emb_im, cs_tab)


def kernel(head_index, rel_type, tail_index, node_emb, node_emb_im, rel_emb):
    cs_tab = _make_cs_table(rel_emb)
    return _sc_score(head_index, rel_type, tail_index,
                     node_emb, node_emb_im, cs_tab)


# speedup vs baseline: 2.7420x; 1.0660x over previous
import functools

import jax
import jax.numpy as jnp
from jax import lax
from jax.experimental import pallas as pl
from jax.experimental.pallas import tpu as pltpu, tpu_sc as plsc

_NC = 2
_NS = 16
_NW = _NC * _NS
_LANES = 16
_CHUNK = 64


def _trig_body(theta_ref, cs_ref):
    th = theta_ref[...]
    cb = lax.bitcast_convert_type(jnp.cos(th).astype(jnp.bfloat16),
                                  jnp.uint16).astype(jnp.int32)
    sb = lax.bitcast_convert_type(jnp.sin(th).astype(jnp.bfloat16),
                                  jnp.uint16).astype(jnp.int32)
    cs_ref[...] = lax.shift_left(cb, 16) | sb


def _make_cs_table(rel_emb):
    r, h = rel_emb.shape
    return pl.pallas_call(
        _trig_body,
        out_shape=jax.ShapeDtypeStruct((r, h), jnp.int32),
    )(rel_emb)


def _sc_score(head_index, rel_type, tail_index, node_emb, node_emb_im, cs_tab):
    batch = head_index.shape[0]
    hidden = node_emb.shape[1]
    nslice = hidden // _LANES
    per_w = batch // _NW
    n_chunks = per_w // _CHUNK
    mesh = plsc.VectorSubcoreMesh(
        core_axis_name="c", subcore_axis_name="s",
        num_cores=_NC, num_subcores=_NS,
    )

    @functools.partial(
        pl.kernel,
        out_type=jax.ShapeDtypeStruct((batch,), jnp.float32),
        mesh=mesh,
        scratch_types=[
            pltpu.VMEM((2, _CHUNK), jnp.int32),
            pltpu.VMEM((2, _CHUNK), jnp.int32),
            pltpu.VMEM((2, _CHUNK), jnp.int32),
            pltpu.VMEM((2, _CHUNK, hidden), jnp.float32),
            pltpu.VMEM((2, _CHUNK, hidden), jnp.float32),
            pltpu.VMEM((2, _CHUNK, hidden), jnp.float32),
            pltpu.VMEM((2, _CHUNK, hidden), jnp.float32),
            pltpu.VMEM((2, _CHUNK, hidden), jnp.int32),
            pltpu.VMEM((per_w,), jnp.float32),
            pltpu.SemaphoreType.DMA,
            pltpu.SemaphoreType.DMA,
            pltpu.SemaphoreType.DMA,
            pltpu.SemaphoreType.DMA,
        ],
    )
    def score_kernel(head_hbm, rel_hbm, tail_hbm, emb_hbm, embim_hbm, cs_hbm,
                     out_hbm, idx_h, idx_r, idx_t, hre, him, tre, tim, cs,
                     score, sem_i0, sem_i1, sem_g0, sem_g1):
        sem_i = (sem_i0, sem_i1)
        sem_g = (sem_g0, sem_g1)
        wid = lax.axis_index("s") * _NC + lax.axis_index("c")
        base = wid * per_w
        lane_iota = lax.iota(jnp.int32, _LANES)
        perm_idx = [lax.iota(jnp.int32, _LANES) ^ jnp.int32(d)
                    for d in (1, 2, 4, 8)]
        gdims = lax.GatherDimensionNumbers(
            offset_dims=(), collapsed_slice_dims=(0,), start_index_map=(0,))

        def _lperm(x, pidx):
            return lax.gather(x, pidx[:, None], gdims, (1,),
                              mode=lax.GatherScatterMode.PROMISE_IN_BOUNDS)

        def fire_idx(c):
            p = c & 1
            cbase = base + c * _CHUNK
            return (
                pltpu.async_copy(head_hbm.at[pl.ds(cbase, _CHUNK)],
                                 idx_h.at[p], sem_i[p]),
                pltpu.async_copy(rel_hbm.at[pl.ds(cbase, _CHUNK)],
                                 idx_r.at[p], sem_i[p]),
                pltpu.async_copy(tail_hbm.at[pl.ds(cbase, _CHUNK)],
                                 idx_t.at[p], sem_i[p]),
            )

        def fire_gather(c):
            p = c & 1
            return (
                pltpu.async_copy(emb_hbm.at[idx_h.at[p]], hre.at[p], sem_g[p]),
                pltpu.async_copy(embim_hbm.at[idx_h.at[p]], him.at[p], sem_g[p]),
                pltpu.async_copy(emb_hbm.at[idx_t.at[p]], tre.at[p], sem_g[p]),
                pltpu.async_copy(embim_hbm.at[idx_t.at[p]], tim.at[p], sem_g[p]),
                pltpu.async_copy(cs_hbm.at[idx_r.at[p]], cs.at[p], sem_g[p]),
            )

        def drain(cps):
            for cp in cps:
                cp.wait()

        idx_cps = {0: fire_idx(0)}
        drain(idx_cps[0])
        gat_cps = {0: fire_gather(0)}
        if n_chunks > 1:
            idx_cps[1] = fire_idx(1)

        for c in range(n_chunks):
            p = c & 1
            if c + 1 < n_chunks:
                drain(idx_cps[c + 1])
                gat_cps[c + 1] = fire_gather(c + 1)
            drain(gat_cps[c])
            if c + 2 < n_chunks:
                idx_cps[c + 2] = fire_idx(c + 2)

            def gbody(g, carry):
                def tbody(t, res):
                    row = g * _LANES + t
                    acc = jnp.zeros((_LANES,), jnp.float32)
                    for j in range(nslice):
                        sl = pl.ds(j * _LANES, _LANES)
                        w = cs[p, row, sl]
                        cv = lax.bitcast_convert_type(
                            w & jnp.int32(-65536), jnp.float32)
                        sv = lax.bitcast_convert_type(
                            lax.shift_left(w, 16), jnp.float32)
                        a = hre[p, row, sl]
                        b = him[p, row, sl]
                        u = tre[p, row, sl]
                        v = tim[p, row, sl]
                        re = cv * a - sv * b - u
                        im = cv * b + sv * a - v
                        acc = acc + (re * re + im * im)
                    for pidx in perm_idx:
                        acc = acc + _lperm(acc, pidx)
                    return jnp.where(lane_iota == t, acc, res)

                s2 = lax.fori_loop(0, _LANES, tbody,
                                   jnp.zeros((_LANES,), jnp.float32))
                x = jnp.maximum(s2, jnp.float32(1e-12))
                bits = lax.bitcast_convert_type(x, jnp.int32)
                bits = jnp.int32(0x5F3759DF) - lax.shift_right_logical(bits, 1)
                y = lax.bitcast_convert_type(bits, jnp.float32)
                for _ in range(3):
                    y = y * (jnp.float32(1.5) - jnp.float32(0.5) * x * y * y)
                score[pl.ds(c * _CHUNK + g * _LANES, _LANES)] = -(x * y)
                return carry

            lax.fori_loop(0, _CHUNK // _LANES, gbody, 0)

        pltpu.sync_copy(score, out_hbm.at[pl.ds(base, per_w)])

    return score_kernel(head_index, rel_type, tail_index,
                        node_emb, node_emb_im, cs_tab)


def kernel(head_index, rel_type, tail_index, node_emb, node_emb_im, rel_emb):
    cs_tab = _make_cs_table(rel_emb)
    return _sc_score(head_index, rel_type, tail_index,
                     node_emb, node_emb_im, cs_tab)
